# Initial kernel scaffold; baseline (speedup 1.0000x reference)
import jax
import jax.numpy as jnp
from jax.experimental import pallas as pl


def kernel(output, target):
    raise NotImplementedError("write your pallas kernel here")


# speedup vs baseline: 2.3916x; 2.3916x over previous
import functools

import jax
import jax.numpy as jnp
import numpy as np
from jax.experimental import pallas as pl
from jax.experimental.pallas import tpu as pltpu

_LABEL_SMOOTHING = 0.1
_V = 100000
_B = 1024
_IGNORE = 1
_S = np.float32(_LABEL_SMOOTHING / (_V - 2))
_C = np.float32(1.0 - _LABEL_SMOOTHING)
_K = float(_V - 2) * float(np.float32(_S * np.float32(np.log(_S)))) + float(
    np.float32(_C * np.float32(np.log(_C)))
)

_BC = 2048
_NB = -(-_V // _BC)


def _body(x_ref, t_ref, o_ref, acc_ref, g_ref, x1_ref):
    j = pl.program_id(0)

    @pl.when(j == 0)
    def _init():
        acc_ref[...] = jnp.zeros_like(acc_ref)
        g_ref[...] = jnp.zeros_like(g_ref)
        x1_ref[...] = x_ref[:, _IGNORE : _IGNORE + 1]

    t = t_ref[...]
    cols = j * _BC + jax.lax.broadcasted_iota(jnp.int32, (_B, _BC), 1)

    def _accum(x):
        acc_ref[...] += jnp.sum(x, axis=1, keepdims=True)
        g_ref[...] += jnp.sum(jnp.where(cols == t, x, 0.0), axis=1, keepdims=True)

    @pl.when(j < _NB - 1)
    def _full():
        _accum(x_ref[...])

    @pl.when(j == _NB - 1)
    def _last():
        _accum(jnp.where(cols < _V, x_ref[...], 0.0))
        maskf = (t != _IGNORE).astype(jnp.float32)
        per_row = maskf * (
            _K + _S * x1_ref[...] - (_C - _S) * g_ref[...] - _S * acc_ref[...]
        )
        o_ref[...] = jnp.sum(per_row, keepdims=True).reshape(1, 1)


@functools.partial(jax.jit)
def kernel(output, target):
    t2 = target.astype(jnp.int32).reshape(_B, 1)
    res = pl.pallas_call(
        _body,
        grid=(_NB,),
        in_specs=[
            pl.BlockSpec((_B, _BC), lambda j: (0, j)),
            pl.BlockSpec((_B, 1), lambda j: (0, 0)),
        ],
        out_specs=pl.BlockSpec((1, 1), lambda j: (0, 0)),
        out_shape=jax.ShapeDtypeStruct((1, 1), jnp.float32),
        scratch_shapes=[
            pltpu.VMEM((_B, 1), jnp.float32),
            pltpu.VMEM((_B, 1), jnp.float32),
            pltpu.VMEM((_B, 1), jnp.float32),
        ],
    )(output, t2)
    return res[0, 0]
